# Initial kernel scaffold; baseline (speedup 1.0000x reference)
import jax
import jax.numpy as jnp
from jax.experimental import pallas as pl


def kernel(idx, A, B):
    raise NotImplementedError("write your pallas kernel here")


# speedup vs baseline: 4.1636x; 4.1636x over previous
import functools

import jax
import jax.numpy as jnp
from jax import lax
from jax.experimental import pallas as pl
from jax.experimental.pallas import tpu as pltpu
from jax.experimental.pallas import tpu_sc as plsc

NUM_MODELS = 4
VOCAB = 100000
RANK = 8
DIM = 32
SEQ = 1024 * 20
NTOT = NUM_MODELS * SEQ

NC, NS = 2, 16
NW = NC * NS
B_PER_W = NTOT // NW


def _sc_gather(gidx, table):
    mesh = plsc.VectorSubcoreMesh(core_axis_name="c", subcore_axis_name="s")

    @functools.partial(
        pl.kernel,
        mesh=mesh,
        compiler_params=pltpu.CompilerParams(use_tc_tiling_on_sc=False),
        out_type=jax.ShapeDtypeStruct((NTOT, RANK), jnp.float32),
        scratch_types=[
            pltpu.VMEM((B_PER_W,), jnp.int32),
            pltpu.VMEM((B_PER_W, RANK), jnp.float32),
            pltpu.SemaphoreType.DMA,
        ],
    )
    def gather_kernel(idx_hbm, tab_hbm, o_hbm, idx_v, rows_v, sem):
        wid = lax.axis_index("s") * NC + lax.axis_index("c")
        base = wid * B_PER_W
        pltpu.sync_copy(idx_hbm.at[pl.ds(base, B_PER_W)], idx_v)
        pltpu.async_copy(tab_hbm.at[idx_v], rows_v, sem).wait()
        pltpu.sync_copy(rows_v, o_hbm.at[pl.ds(base, B_PER_W)])

    return gather_kernel(gidx, table)


def _tc_matmul(g, B):
    def body(g_ref, b_ref, o_ref):
        o_ref[0] = jnp.dot(g_ref[0], b_ref[0],
                           preferred_element_type=jnp.float32)

    return pl.pallas_call(
        body,
        grid=(NUM_MODELS,),
        in_specs=[
            pl.BlockSpec((1, SEQ, RANK), lambda m: (m, 0, 0)),
            pl.BlockSpec((1, RANK, DIM), lambda m: (m, 0, 0)),
        ],
        out_specs=pl.BlockSpec((1, SEQ, DIM), lambda m: (m, 0, 0)),
        out_shape=jax.ShapeDtypeStruct((NUM_MODELS, SEQ, DIM), jnp.float32),
    )(g, B)


def kernel(idx, A, B):
    m, b, t = idx.shape
    offs = (jnp.arange(NUM_MODELS, dtype=jnp.int32) * VOCAB)[:, None, None]
    gidx = (idx.astype(jnp.int32) + offs).reshape(-1)
    table = A.reshape(NUM_MODELS * VOCAB, RANK)
    g = _sc_gather(gidx, table)
    out = _tc_matmul(g.reshape(NUM_MODELS, SEQ, RANK), B)
    return out.reshape(m, b, t, DIM)
